# Initial kernel scaffold; baseline (speedup 1.0000x reference)
import jax
import jax.numpy as jnp
from jax.experimental import pallas as pl


def kernel(hyperedge_features, text_embedding, W_proj):
    raise NotImplementedError("write your pallas kernel here")


# speedup vs baseline: 1.0095x; 1.0095x over previous
import functools

import jax
import jax.numpy as jnp
from jax import lax
from jax.experimental import pallas as pl
from jax.experimental.pallas import tpu as pltpu
from jax.experimental.pallas import tpu_sc as plsc

K = 100000
D = 512
TOP_K = 3
THRESHOLD = 0.5

NC = 2
NS = 16
NW = NC * NS
RPW = K // NW
CH = 48
NFULL = RPW // CH
TAIL_ROW = RPW - CH
NCH = NFULL + 1
RPW_PAD = NCH * CH


def _project_body(text_ref, w_ref, out_ref):
    t = jnp.dot(text_ref[...], w_ref[...], preferred_element_type=jnp.float32)
    n = jnp.sqrt(jnp.sum(t * t))
    out_ref[...] = t / (n + 1e-8)


def _project(text_embedding, W_proj):
    return pl.pallas_call(
        _project_body,
        out_shape=jax.ShapeDtypeStruct((1, D), jnp.float32),
    )(text_embedding, W_proj)


def _chunk_row(c):
    return jnp.minimum(c * CH, TAIL_ROW)


def _sc_body(h_hbm, t_hbm, dot_hbm, ss_hbm,
             t_v, buf0, buf1, dot_stage, ss_stage, sem0, sem1):
    wid = lax.axis_index("s") * NC + lax.axis_index("c")
    base = wid * RPW

    pltpu.sync_copy(t_hbm, t_v)
    lanes = lax.broadcasted_iota(jnp.int32, (16,), 0)
    zeros = jnp.zeros((16,), jnp.float32)

    def compute_chunk(buf, pos):
        def group_body(g, carry):
            def oct_body(o, accs):
                dot_acc, ss_acc = accs
                row0 = g * 16 + o * 8
                d = [zeros] * 8
                s = [zeros] * 8
                for j in range(D // 16):
                    t_j = t_v[pl.ds(j * 16, 16)]
                    for r in range(8):
                        v = buf[row0 + r, pl.ds(j * 16, 16)]
                        d[r] = d[r] + v * t_j
                        s[r] = s[r] + v * v
                for r in range(8):
                    lane = lanes == (o * 8 + r)
                    dot_acc = jnp.where(lane, jnp.sum(d[r]), dot_acc)
                    ss_acc = jnp.where(lane, jnp.sum(s[r]), ss_acc)
                return dot_acc, ss_acc

            dot_acc, ss_acc = lax.fori_loop(0, 2, oct_body, (zeros, zeros))
            dot_stage[pl.ds(pos + g * 16, 16)] = dot_acc
            ss_stage[pl.ds(pos + g * 16, 16)] = ss_acc
            return carry

        lax.fori_loop(0, CH // 16, group_body, 0)

    def start(c, buf, sem):
        pltpu.async_copy(h_hbm.at[pl.ds(base + _chunk_row(c), CH)], buf, sem)

    def wait(buf, sem):
        pltpu.make_async_copy(h_hbm.at[pl.ds(0, CH)], buf, sem).wait()

    start(0, buf0, sem0)
    start(1, buf1, sem1)

    def pair_body(p, carry):
        c = 2 * p
        wait(buf0, sem0)
        compute_chunk(buf0, c * CH)
        start(c + 2, buf0, sem0)
        wait(buf1, sem1)
        compute_chunk(buf1, (c + 1) * CH)
        start(c + 3, buf1, sem1)
        return carry

    lax.fori_loop(0, NCH // 2 - 1, pair_body, 0)

    wait(buf0, sem0)
    compute_chunk(buf0, (NCH - 2) * CH)
    wait(buf1, sem1)
    compute_chunk(buf1, (NCH - 1) * CH)

    pltpu.sync_copy(dot_stage, dot_hbm.at[wid])
    pltpu.sync_copy(ss_stage, ss_hbm.at[wid])


def _sc_stream(h, t_n):
    mesh = plsc.VectorSubcoreMesh(
        core_axis_name="c", subcore_axis_name="s",
        num_cores=NC, num_subcores=NS)
    f = pl.kernel(
        _sc_body,
        out_type=(
            jax.ShapeDtypeStruct((NW, RPW_PAD), jnp.float32),
            jax.ShapeDtypeStruct((NW, RPW_PAD), jnp.float32),
        ),
        mesh=mesh,
        compiler_params=pltpu.CompilerParams(
            use_tc_tiling_on_sc=False, needs_layout_passes=False),
        scratch_types=[
            pltpu.VMEM((D,), jnp.float32),
            pltpu.VMEM((CH, D), jnp.float32),
            pltpu.VMEM((CH, D), jnp.float32),
            pltpu.VMEM((RPW_PAD,), jnp.float32),
            pltpu.VMEM((RPW_PAD,), jnp.float32),
            pltpu.SemaphoreType.DMA,
            pltpu.SemaphoreType.DMA,
        ],
    )
    return f(h, t_n)


def _topk_body(dot_ref, ss_ref, s_ref, i_ref, v_ref):
    d = dot_ref[...]
    ss = ss_ref[...]
    sim = d / (jnp.sqrt(ss) + 1e-8)
    w = lax.broadcasted_iota(jnp.int32, (NW, RPW_PAD), 0)
    p = lax.broadcasted_iota(jnp.int32, (NW, RPW_PAD), 1)
    row = w * RPW + jnp.where(p < NFULL * CH, p, TAIL_ROW + (p - NFULL * CH))
    neg = jnp.float32(-jnp.inf)

    scores = []
    idxs = []
    for _ in range(TOP_K):
        m = jnp.max(sim)
        eq = sim == m
        idx = jnp.min(jnp.where(eq, row, jnp.int32(2**31 - 1)))
        scores.append(m)
        idxs.append(idx)
        sim = jnp.where(row == idx, neg, sim)

    lane = lax.broadcasted_iota(jnp.int32, (1, 128), 1)
    s_vec = jnp.full((1, 128), 0.0, jnp.float32)
    i_vec = jnp.full((1, 128), 0, jnp.int32)
    for t in range(TOP_K):
        s_vec = jnp.where(lane == t, scores[t], s_vec)
        i_vec = jnp.where(lane == t, idxs[t], i_vec)
    s_ref[...] = s_vec
    i_ref[...] = i_vec
    v_ref[...] = jnp.where(scores[0] >= THRESHOLD, 1, 0) * jnp.ones(
        (1, 128), jnp.int32)


def _topk(dot, ss):
    return pl.pallas_call(
        _topk_body,
        out_shape=(
            jax.ShapeDtypeStruct((1, 128), jnp.float32),
            jax.ShapeDtypeStruct((1, 128), jnp.int32),
            jax.ShapeDtypeStruct((1, 128), jnp.int32),
        ),
    )(dot, ss)


def kernel(hyperedge_features, text_embedding, W_proj):
    t_n = _project(text_embedding, W_proj)
    dot, ss = _sc_stream(hyperedge_features, t_n.reshape(D))
    s, i, v = _topk(dot, ss)
    return s[0, :TOP_K], i[0, :TOP_K], v[0, 0] != 0
